# Initial kernel scaffold; baseline (speedup 1.0000x reference)
import jax
import jax.numpy as jnp
from jax.experimental import pallas as pl


def kernel(x, w1_weight, w1_bias, router_bias):
    raise NotImplementedError("write your pallas kernel here")


# speedup vs baseline: 2.4548x; 2.4548x over previous
import functools

import jax
import jax.numpy as jnp
from jax import lax
from jax.experimental import pallas as pl
from jax.experimental.pallas import tpu as pltpu
from jax.experimental.pallas import tpu_sc as plsc

_TOKENS = 8192
_DIM = 2048
_NE = 64
_KG = 4
_TK = 8
_SCALE = 2.5
_NEG = jnp.float32(-jnp.inf)
_NC = 2
_NS = 16


def _dense_body(x_ref, wt_ref, b_ref, rb_ref, s_ref):
    logits = jnp.dot(x_ref[...], wt_ref[...],
                     preferred_element_type=jnp.float32)
    sig = jax.nn.sigmoid(logits + b_ref[...])
    s_ref[...] = sig / jnp.sum(sig, axis=-1, keepdims=True) + rb_ref[...]


def _dense_scores(x, wt, b, rb):
    blk = 512
    return pl.pallas_call(
        _dense_body,
        grid=(_TOKENS // blk,),
        in_specs=[
            pl.BlockSpec((blk, _DIM), lambda i: (i, 0)),
            pl.BlockSpec((_DIM, _NE), lambda i: (0, 0)),
            pl.BlockSpec((1, _NE), lambda i: (0, 0)),
            pl.BlockSpec((1, _NE), lambda i: (0, 0)),
        ],
        out_specs=pl.BlockSpec((blk, _NE), lambda i: (i, 0)),
        out_shape=jax.ShapeDtypeStruct((_TOKENS, _NE), jnp.float32),
        compiler_params=pltpu.CompilerParams(
            dimension_semantics=("arbitrary",)),
    )(x, wt, b, rb)


_SORT8 = [(0, 1), (2, 3), (4, 5), (6, 7), (0, 2), (1, 3), (4, 6), (5, 7),
          (1, 2), (5, 6), (0, 4), (1, 5), (2, 6), (3, 7), (2, 4), (3, 5),
          (1, 2), (3, 4), (5, 6)]


def _route_token(sbuf, t, lane, eids, lo_half):
    s = [sbuf[t, pl.ds(16 * i, 16)] for i in range(4)]
    gmax = []
    for v in s:
        gmax.append(jnp.max(jnp.where(lo_half, v, _NEG)))
        gmax.append(jnp.max(jnp.where(lo_half, _NEG, v)))
    g = list(gmax)
    for i, j in _SORT8:
        g[i], g[j] = jnp.minimum(g[i], g[j]), jnp.maximum(g[i], g[j])
    thr = g[4]
    ms = []
    for i, v in enumerate(s):
        gsel = jnp.where(lo_half, gmax[2 * i], gmax[2 * i + 1])
        ms.append(jnp.where(gsel >= thr, v, _NEG))
    return ms


def _top8_step(ms, eids, mxb):
    cand = jnp.full((16,), _NE, jnp.int32)
    for v, e in zip(ms, eids):
        cand = jnp.minimum(cand, jnp.where(v == mxb, e, _NE))
    return jnp.min(cand)


def _routing(scores):
    nw = _NC * _NS
    tpw = _TOKENS // nw
    mesh = plsc.VectorSubcoreMesh(core_axis_name="c", subcore_axis_name="s")

    @functools.partial(
        pl.kernel,
        mesh=mesh,
        out_type=[jax.ShapeDtypeStruct((_TOKENS * _TK,), jnp.float32),
                  jax.ShapeDtypeStruct((_TOKENS * _TK,), jnp.int32)],
        scratch_types=[pltpu.VMEM((tpw, _NE), jnp.float32),
                       pltpu.VMEM((tpw * _TK,), jnp.float32),
                       pltpu.VMEM((tpw * _TK,), jnp.int32)],
        compiler_params=pltpu.CompilerParams(needs_layout_passes=False),
    )
    def body(scores_hbm, vals_hbm, idx_hbm, sbuf, vbuf, ibuf):
        wid = lax.axis_index("s") * _NC + lax.axis_index("c")
        base = wid * tpw
        pltpu.sync_copy(scores_hbm.at[pl.ds(base, tpw)], sbuf)
        lane = lax.iota(jnp.int32, 16)
        lo_half = lane < 8
        eids = [lane + 16 * i for i in range(4)]

        def pair(p, carry):
            ms0 = _route_token(sbuf, 2 * p, lane, eids, lo_half)
            ms1 = _route_token(sbuf, 2 * p + 1, lane, eids, lo_half)
            outv = jnp.zeros((16,), jnp.float32)
            outi = jnp.zeros((16,), jnp.int32)
            for k in range(_TK):
                mx0 = jnp.max(jnp.maximum(jnp.maximum(ms0[0], ms0[1]),
                                          jnp.maximum(ms0[2], ms0[3])))
                mx1 = jnp.max(jnp.maximum(jnp.maximum(ms1[0], ms1[1]),
                                          jnp.maximum(ms1[2], ms1[3])))
                mxb0 = jnp.full((16,), mx0)
                mxb1 = jnp.full((16,), mx1)
                i0 = _top8_step(ms0, eids, mxb0)
                i1 = _top8_step(ms1, eids, mxb1)
                outv = jnp.where(lane == k, mx0 * _SCALE, outv)
                outv = jnp.where(lane == 8 + k, mx1 * _SCALE, outv)
                outi = jnp.where(lane == k, i0, outi)
                outi = jnp.where(lane == 8 + k, i1, outi)
                ib0 = jnp.full((16,), i0)
                ib1 = jnp.full((16,), i1)
                ms0 = [jnp.where(e == ib0, _NEG, v)
                       for v, e in zip(ms0, eids)]
                ms1 = [jnp.where(e == ib1, _NEG, v)
                       for v, e in zip(ms1, eids)]
            vbuf[pl.ds(16 * p, 16)] = outv
            ibuf[pl.ds(16 * p, 16)] = outi
            return carry

        lax.fori_loop(0, tpw // 2, pair, 0)
        pltpu.sync_copy(vbuf, vals_hbm.at[pl.ds(base * _TK, tpw * _TK)])
        pltpu.sync_copy(ibuf, idx_hbm.at[pl.ds(base * _TK, tpw * _TK)])

    return body(scores)


def kernel(x, w1_weight, w1_bias, router_bias):
    scores = _dense_scores(x, w1_weight.T, w1_bias.reshape(1, _NE),
                           router_bias.reshape(1, _NE))
    vals, ids = _routing(scores)
    return vals.reshape(_TOKENS, _TK), ids.reshape(_TOKENS, _TK)
